# baseline (speedup 1.0000x reference)
import functools

import jax
import jax.numpy as jnp
from jax import lax
from jax.experimental import pallas as pl
from jax.experimental.pallas import tpu as pltpu
from jax.experimental.pallas import tpu_sc as plsc

B = 16384
N_NUM = 13
N_CAT = 26
VOCAB = 100000
D = 64
N_TOK = 1 + N_NUM + N_CAT
LANES = 16
NR = D // LANES

NC = 2
NS = 16
NW = NC * NS
RPW = B // NW
CB = 16
NCHUNK = RPW // CB


def _tokenizer_body(xnum_hbm, idx_hbm, w_hbm, bias_hbm, cls_hbm, tbl_hbm,
                    out_hbm, xnum_v, idx_v, w_v, bias_v, cls_v, out_v, gsem):
    wid = lax.axis_index("s") * NC + lax.axis_index("c")
    row0 = wid * RPW

    pltpu.sync_copy(xnum_hbm.at[pl.ds(row0 * LANES, RPW * LANES)], xnum_v)
    pltpu.sync_copy(w_hbm, w_v)
    pltpu.sync_copy(bias_hbm, bias_v)
    pltpu.sync_copy(cls_hbm, cls_v)
    cls_r = [cls_v[pl.ds(LANES * r, LANES)] for r in range(NR)]

    def chunk_body(c, carry):
        base = row0 + c * CB
        pltpu.sync_copy(idx_hbm.at[pl.ds(base, CB)], idx_v)

        handles = []
        for b in range(CB):
            handles.append(pltpu.async_copy(
                tbl_hbm.at[idx_v.at[b]],
                out_v.at[b, pl.ds(1 + N_NUM, N_CAT)],
                gsem))

        def cls_body(b, carry2):
            for r in range(NR):
                out_v[b, 0, pl.ds(LANES * r, LANES)] = cls_r[r]
            return carry2
        lax.fori_loop(0, CB, cls_body, 0, unroll=4)

        for i in range(N_NUM):
            wr = [w_v[i, pl.ds(LANES * r, LANES)] for r in range(NR)]
            br = [bias_v[i, pl.ds(LANES * r, LANES)] for r in range(NR)]

            def num_body(b, carry2, i=i, wr=wr, br=br):
                xv = xnum_v[pl.ds((c * CB + b) * LANES, LANES)]
                xs = xv[i]
                for r in range(NR):
                    out_v[b, 1 + i, pl.ds(LANES * r, LANES)] = xs * wr[r] + br[r]
                return carry2
            lax.fori_loop(0, CB, num_body, 0, unroll=4)

        for h in handles:
            h.wait()
        pltpu.sync_copy(out_v, out_hbm.at[pl.ds(base, CB)])
        return carry

    lax.fori_loop(0, NCHUNK, chunk_body, 0)


@jax.jit
def _tokenizer(xnum_flat, idx, weight, bias, cls_flat, tbl_flat):
    mesh = plsc.VectorSubcoreMesh(core_axis_name="c", subcore_axis_name="s")
    kern = pl.kernel(
        _tokenizer_body,
        out_type=jax.ShapeDtypeStruct((B, N_TOK, D), jnp.float32),
        mesh=mesh,
        scratch_types=[
            pltpu.VMEM((RPW * LANES,), jnp.float32),
            pltpu.VMEM((CB, N_CAT), jnp.int32),
            pltpu.VMEM((N_NUM, D), jnp.float32),
            pltpu.VMEM((N_NUM, D), jnp.float32),
            pltpu.VMEM((D,), jnp.float32),
            pltpu.VMEM((CB, N_TOK, D), jnp.float32),
            pltpu.SemaphoreType.DMA,
        ],
        compiler_params=pltpu.CompilerParams(use_tc_tiling_on_sc=False),
    )
    return kern(xnum_flat, idx, weight, bias, cls_flat, tbl_flat)


def kernel(x_num, x_cat, weight, bias, cls_token, tables):
    idx = x_cat.astype(jnp.int32) + (
        jnp.arange(N_CAT, dtype=jnp.int32) * VOCAB)[None, :]
    x_num_pad = jnp.pad(x_num, ((0, 0), (0, LANES - N_NUM)))
    return _tokenizer(
        x_num_pad.reshape(B * LANES),
        idx,
        weight,
        bias,
        cls_token.reshape(D),
        tables.reshape(N_CAT * VOCAB, D),
    )
